# baseline (speedup 1.0000x reference)
import functools

import jax
import jax.numpy as jnp
from jax.experimental import pallas as pl
from jax.experimental.pallas import tpu as pltpu

_EPS = 1e-7


def _loss_kernel(ts_ref, os_ref, td_ref, od_ref, out_ref, acc_ref, *, n_rows):
    i = pl.program_id(0)
    g = pl.num_programs(0)

    ts = ts_ref[...]
    osc = os_ref[...]
    td = td_ref[...]
    od = od_ref[...]

    s = jnp.sum(osc, axis=1, keepdims=True)
    p = jnp.clip(osc / s, _EPS, 1.0 - _EPS)
    ce_c = jnp.sum(ts * jnp.log(p))

    col = jax.lax.broadcasted_iota(jnp.int32, ts.shape, 1)
    fg = (col >= 1).astype(jnp.float32)
    pos_c = jnp.sum(ts * fg)
    mask = jnp.where(ts == 1.0, fg, 0.0)

    x = td - od
    ax = jnp.abs(x)
    h = jnp.where(ax < 1.0, 0.5 * x * x, ax - 0.5)
    nd = td.shape[1]
    nc = ts.shape[1]
    er = jax.lax.broadcasted_iota(jnp.int32, (nd, nc), 0)
    ec = jax.lax.broadcasted_iota(jnp.int32, (nd, nc), 1)
    fold = ((er // 4) == ec).astype(jnp.float32)
    hsum = jax.lax.dot_general(
        h, fold, (((1,), (0,)), ((), ())),
        preferred_element_type=jnp.float32,
        precision=jax.lax.Precision.HIGHEST,
    )
    reg_c = jnp.sum(hsum * mask)

    @pl.when(i == 0)
    def _init():
        acc_ref[0] = 0.0
        acc_ref[1] = 0.0
        acc_ref[2] = 0.0

    acc_ref[0] += ce_c
    acc_ref[1] += pos_c
    acc_ref[2] += reg_c

    @pl.when(i == g - 1)
    def _fin():
        cls_loss = -acc_ref[0] / n_rows
        reg_loss = acc_ref[2] / jnp.maximum(_EPS, acc_ref[1])
        out_ref[...] = jnp.reshape(cls_loss + reg_loss, (1, 1))


@jax.jit
def kernel(target_deltas, target_scores, output_deltas, output_scores):
    b, n, c = target_scores.shape
    d = target_deltas.shape[-1]
    rows = b * n
    ts2 = target_scores.reshape(rows, c)
    os2 = output_scores.reshape(rows, c)
    td2 = target_deltas.reshape(rows, d)
    od2 = output_deltas.reshape(rows, d)

    blk = 2000
    grid = rows // blk

    out = pl.pallas_call(
        functools.partial(_loss_kernel, n_rows=float(rows)),
        grid=(grid,),
        in_specs=[
            pl.BlockSpec((blk, c), lambda i: (i, 0)),
            pl.BlockSpec((blk, c), lambda i: (i, 0)),
            pl.BlockSpec((blk, d), lambda i: (i, 0)),
            pl.BlockSpec((blk, d), lambda i: (i, 0)),
        ],
        out_specs=pl.BlockSpec((1, 1), lambda i: (0, 0)),
        out_shape=jax.ShapeDtypeStruct((1, 1), jnp.float32),
        scratch_shapes=[pltpu.SMEM((3,), jnp.float32)],
    )(ts2, os2, td2, od2)
    return out[0, 0]


# speedup vs baseline: 3.6200x; 3.6200x over previous
import functools

import jax
import jax.numpy as jnp
from jax.experimental import pallas as pl
from jax.experimental.pallas import tpu as pltpu

_EPS = 1e-7


def _loss_kernel(ts_ref, os_ref, td_ref, od_ref, out_ref, acc_ref, *, n_rows):
    i = pl.program_id(0) * pl.num_programs(1) + pl.program_id(1)
    g = pl.num_programs(0) * pl.num_programs(1)

    ts = ts_ref[0]
    osc = os_ref[0]
    td = td_ref[0]
    od = od_ref[0]

    s = jnp.sum(osc, axis=1, keepdims=True)
    p = jnp.clip(osc / s, _EPS, 1.0 - _EPS)
    ce_c = jnp.sum(ts * jnp.log(p))

    col = jax.lax.broadcasted_iota(jnp.int32, ts.shape, 1)
    fg = (col >= 1).astype(jnp.float32)
    pos_c = jnp.sum(ts * fg)
    mask = jnp.where(ts == 1.0, fg, 0.0)

    x = td - od
    ax = jnp.abs(x)
    h = jnp.where(ax < 1.0, 0.5 * x * x, ax - 0.5)
    nd = td.shape[1]
    nc = ts.shape[1]
    er = jax.lax.broadcasted_iota(jnp.int32, (nd, nc), 0)
    ec = jax.lax.broadcasted_iota(jnp.int32, (nd, nc), 1)
    fold = ((er // 4) == ec).astype(jnp.float32)
    hsum = jax.lax.dot_general(
        h, fold, (((1,), (0,)), ((), ())),
        preferred_element_type=jnp.float32,
        precision=jax.lax.Precision.HIGHEST,
    )
    reg_c = jnp.sum(hsum * mask)

    @pl.when(i == 0)
    def _init():
        acc_ref[0] = 0.0
        acc_ref[1] = 0.0
        acc_ref[2] = 0.0

    acc_ref[0] += ce_c
    acc_ref[1] += pos_c
    acc_ref[2] += reg_c

    @pl.when(i == g - 1)
    def _fin():
        cls_loss = -acc_ref[0] / n_rows
        reg_loss = acc_ref[2] / jnp.maximum(_EPS, acc_ref[1])
        out_ref[...] = jnp.reshape(cls_loss + reg_loss, (1, 1))


@jax.jit
def kernel(target_deltas, target_scores, output_deltas, output_scores):
    b, n, c = target_scores.shape
    d = target_deltas.shape[-1]
    rows = b * n

    blk = 2000
    grid = (b, n // blk)

    out = pl.pallas_call(
        functools.partial(_loss_kernel, n_rows=float(rows)),
        grid=grid,
        in_specs=[
            pl.BlockSpec((1, blk, c), lambda i, j: (i, j, 0)),
            pl.BlockSpec((1, blk, c), lambda i, j: (i, j, 0)),
            pl.BlockSpec((1, blk, d), lambda i, j: (i, j, 0)),
            pl.BlockSpec((1, blk, d), lambda i, j: (i, j, 0)),
        ],
        out_specs=pl.BlockSpec((1, 1), lambda i, j: (0, 0)),
        out_shape=jax.ShapeDtypeStruct((1, 1), jnp.float32),
        scratch_shapes=[pltpu.SMEM((3,), jnp.float32)],
    )(target_scores, output_scores, target_deltas, output_deltas)
    return out[0, 0]


# speedup vs baseline: 4.0062x; 1.1067x over previous
import functools

import jax
import jax.numpy as jnp
from jax.experimental import pallas as pl
from jax.experimental.pallas import tpu as pltpu

_EPS = 1e-7


def _loss_kernel(ts_ref, os_ref, td_ref, od_ref, out_ref, acc_ref, *, n_rows):
    i = pl.program_id(0) * pl.num_programs(1) + pl.program_id(1)
    g = pl.num_programs(0) * pl.num_programs(1)

    ts = ts_ref[0]
    osc = os_ref[0]
    td = td_ref[0]
    od = od_ref[0]

    s = jnp.sum(osc, axis=1, keepdims=True)
    p = jnp.clip(osc / s, _EPS, 1.0 - _EPS)
    ce_c = jnp.sum(ts * jnp.log(p))

    col = jax.lax.broadcasted_iota(jnp.int32, ts.shape, 1)
    fg = (col >= 1).astype(jnp.float32)
    pos_c = jnp.sum(ts * fg)
    mask = jnp.where(ts == 1.0, fg, 0.0)

    x = td - od
    ax = jnp.abs(x)
    h = jnp.where(ax < 1.0, 0.5 * x * x, ax - 0.5)
    nd = td.shape[1]
    nc = ts.shape[1]
    er = jax.lax.broadcasted_iota(jnp.int32, (nd, nc), 0)
    ec = jax.lax.broadcasted_iota(jnp.int32, (nd, nc), 1)
    fold = ((er // 4) == ec).astype(jnp.float32)
    hsum = jax.lax.dot_general(
        h, fold, (((1,), (0,)), ((), ())),
        preferred_element_type=jnp.float32,
        precision=jax.lax.Precision.DEFAULT,
    )
    reg_c = jnp.sum(hsum * mask)

    @pl.when(i == 0)
    def _init():
        acc_ref[0] = 0.0
        acc_ref[1] = 0.0
        acc_ref[2] = 0.0

    acc_ref[0] += ce_c
    acc_ref[1] += pos_c
    acc_ref[2] += reg_c

    @pl.when(i == g - 1)
    def _fin():
        cls_loss = -acc_ref[0] / n_rows
        reg_loss = acc_ref[2] / jnp.maximum(_EPS, acc_ref[1])
        out_ref[...] = jnp.reshape(cls_loss + reg_loss, (1, 1))


@jax.jit
def kernel(target_deltas, target_scores, output_deltas, output_scores):
    b, n, c = target_scores.shape
    d = target_deltas.shape[-1]
    rows = b * n

    blk = 3000
    grid = (b, n // blk)

    out = pl.pallas_call(
        functools.partial(_loss_kernel, n_rows=float(rows)),
        grid=grid,
        in_specs=[
            pl.BlockSpec((1, blk, c), lambda i, j: (i, j, 0)),
            pl.BlockSpec((1, blk, c), lambda i, j: (i, j, 0)),
            pl.BlockSpec((1, blk, d), lambda i, j: (i, j, 0)),
            pl.BlockSpec((1, blk, d), lambda i, j: (i, j, 0)),
        ],
        out_specs=pl.BlockSpec((1, 1), lambda i, j: (0, 0)),
        out_shape=jax.ShapeDtypeStruct((1, 1), jnp.float32),
        scratch_shapes=[pltpu.SMEM((3,), jnp.float32)],
    )(target_scores, output_scores, target_deltas, output_deltas)
    return out[0, 0]
